# Initial kernel scaffold; baseline (speedup 1.0000x reference)
import jax
import jax.numpy as jnp
from jax.experimental import pallas as pl


def kernel(x, edge_index):
    raise NotImplementedError("write your pallas kernel here")


# speedup vs baseline: 132.3744x; 132.3744x over previous
import jax
import jax.numpy as jnp
from jax.experimental import pallas as pl


def _pool_body(x_ref, e_ref, o1, o2, o3, o4, f1, f2, f3, f4):
    v = x_ref[...]

    def pool(t, groups):
        even = jnp.concatenate(
            [t[..., (2 * j) * 256:(2 * j + 1) * 256] for j in range(groups)],
            axis=-1)
        odd = jnp.concatenate(
            [t[..., (2 * j + 1) * 256:(2 * j + 2) * 256] for j in range(groups)],
            axis=-1)
        return (even + odd) * 0.5

    p1 = pool(v, 8)
    p2 = pool(p1, 4)
    p3 = pool(p2, 2)
    p4 = pool(p3, 1)
    o1[...] = p1
    o2[...] = p2
    o3[...] = p3
    o4[...] = p4

    e = e_ref[...]
    f1[...] = e >> 1
    f2[...] = e >> 2
    f3[...] = e >> 3
    f4[...] = e >> 4


def kernel(x, edge_index):
    xr = x.reshape(5, 125, 4096)
    er = edge_index.reshape(5, 250, 256)

    fspec = lambda shp: pl.BlockSpec((1,) + shp[1:], lambda i: (i, 0, 0))
    outs = pl.pallas_call(
        _pool_body,
        grid=(5,),
        in_specs=[fspec((5, 125, 4096)), fspec((5, 250, 256))],
        out_specs=[
            fspec((5, 125, 2048)), fspec((5, 125, 1024)),
            fspec((5, 125, 512)), fspec((5, 125, 256)),
            fspec((5, 250, 256)), fspec((5, 250, 256)),
            fspec((5, 250, 256)), fspec((5, 250, 256)),
        ],
        out_shape=[
            jax.ShapeDtypeStruct((5, 125, 2048), jnp.float32),
            jax.ShapeDtypeStruct((5, 125, 1024), jnp.float32),
            jax.ShapeDtypeStruct((5, 125, 512), jnp.float32),
            jax.ShapeDtypeStruct((5, 125, 256), jnp.float32),
            jax.ShapeDtypeStruct((5, 250, 256), jnp.int32),
            jax.ShapeDtypeStruct((5, 250, 256), jnp.int32),
            jax.ShapeDtypeStruct((5, 250, 256), jnp.int32),
            jax.ShapeDtypeStruct((5, 250, 256), jnp.int32),
        ],
    )(xr, er)
    p1, p2, p3, p4, f1, f2, f3, f4 = outs

    x1 = p1.reshape(5000, 256)
    x2 = p2.reshape(2500, 256)
    x3 = p3.reshape(1250, 256)
    x4 = p4.reshape(625, 256)
    e1 = f1.reshape(2, 160000)
    e2 = f2.reshape(2, 160000)
    e3 = f3.reshape(2, 160000)
    e4 = f4.reshape(2, 160000)
    return (x, x1, x2, x3, x4, edge_index, e1, e2, e3, e4)
